# baseline (speedup 1.0000x reference)
import functools

import jax
import jax.numpy as jnp
from jax import lax
from jax.experimental import pallas as pl
from jax.experimental.pallas import tpu as pltpu
from jax.experimental.pallas import tpu_sc as plsc

B = 1024
J = 26
C = 1000
ROW = J * C
NW = 32
ROWS_PER_W = B // NW
RB = 8
NBLK = ROWS_PER_W // RB
CHUNK = 29 * 128
NCHUNK = 7
TAIL0 = NCHUNK * CHUNK
TAILW = ROW - TAIL0
NBUF = 3

_mesh = plsc.VectorSubcoreMesh(core_axis_name="c", subcore_axis_name="s")


@functools.partial(
    pl.kernel,
    mesh=_mesh,
    out_type=jax.ShapeDtypeStruct((B, ROW), jnp.int32),
    scratch_types=[
        pltpu.VMEM((ROWS_PER_W * J,), jnp.int32),
        pltpu.VMEM((RB, CHUNK), jnp.int32),
        pltpu.VMEM((RB, CHUNK), jnp.int32),
        pltpu.VMEM((RB, CHUNK), jnp.int32),
        pltpu.VMEM((RB, TAILW), jnp.int32),
        pltpu.VMEM((RB, TAILW), jnp.int32),
        pltpu.VMEM((RB, TAILW), jnp.int32),
        pltpu.VMEM((RB, TAILW), jnp.int32),
        pltpu.SemaphoreType.DMA,
        pltpu.SemaphoreType.DMA,
        pltpu.SemaphoreType.DMA,
        pltpu.SemaphoreType.DMA,
    ],
    compiler_params=pltpu.CompilerParams(needs_layout_passes=False),
)
def _onehot_sc(x_hbm, zeros_hbm, out_hbm, xv,
               buf0, buf1, buf2, tl0, tl1, tl2, tl3, s0, s1, s2, st):
    bufs = (buf0, buf1, buf2)
    tails = (tl0, tl1, tl2, tl3)
    sems = (s0, s1, s2)
    wid = lax.axis_index("s") * 2 + lax.axis_index("c")
    base_row = wid * ROWS_PER_W
    pltpu.sync_copy(x_hbm.at[pl.ds(base_row * J, ROWS_PER_W * J)], xv)
    for b in range(NBUF):
        pltpu.sync_copy(zeros_hbm, bufs[b])
    for t in tails:
        for r in range(RB):
            t[r, :] = jnp.zeros((TAILW,), jnp.int32)

    offs = lax.broadcasted_iota(jnp.int32, (16,), 0) * C
    ones = jnp.full((16,), 1, jnp.int32)
    zeros_v = jnp.zeros((16,), jnp.int32)
    rvs = [jnp.full((16,), r, jnp.int32) for r in range(RB)]

    handles = [None] * NBUF
    restore = [None] * NBUF
    tail_handles = []

    for blk in range(NBLK):
        r0 = base_row + blk * RB
        cols = []
        for r in range(RB):
            k = blk * RB + r
            xa = xv[pl.ds(k * J, 16)]
            xb = xv[pl.ds(k * J + (J - 16), 16)]
            cols.append((xa + offs, xb + offs + (J - 16) * C))

        tbuf = tails[blk]
        for r in range(RB):
            _, cb = cols[r]
            plsc.store_scatter(tbuf, [rvs[r], cb - TAIL0], ones,
                               mask=cb >= TAIL0)
        tail_handles.append(pltpu.async_copy(
            tbuf, out_hbm.at[pl.ds(r0, RB), pl.ds(TAIL0, TAILW)], st))

        for chunk in range(NCHUNK):
            gk = blk * NCHUNK + chunk
            slot = gk % NBUF
            if handles[slot] is not None:
                handles[slot].wait()
                pcols, pc0 = restore[slot]
                for r in range(RB):
                    for c in pcols[r]:
                        m = (c >= pc0) & (c < pc0 + CHUNK)
                        plsc.store_scatter(bufs[slot], [rvs[r], c - pc0],
                                           zeros_v, mask=m)
            c0 = chunk * CHUNK
            for r in range(RB):
                for c in cols[r]:
                    m = (c >= c0) & (c < c0 + CHUNK)
                    plsc.store_scatter(bufs[slot], [rvs[r], c - c0], ones,
                                       mask=m)
            dst = out_hbm.at[pl.ds(r0, RB), pl.ds(c0, CHUNK)]
            handles[slot] = pltpu.async_copy(bufs[slot], dst, sems[slot])
            restore[slot] = (cols, c0)

    for h in handles:
        h.wait()
    for h in tail_handles:
        h.wait()


def kernel(x):
    xf = x.reshape(-1).astype(jnp.int32)
    zeros = jnp.zeros((RB, CHUNK), jnp.int32)
    return _onehot_sc(xf, zeros)
